# baseline (speedup 1.0000x reference)
import functools

import jax
import jax.numpy as jnp
from jax import lax
from jax.experimental import pallas as pl
from jax.experimental.pallas import tpu as pltpu
from jax.experimental.pallas import tpu_sc as plsc

B = 16
L = 32
E = 160000
N = 10000
D = 128

NC = 2
NS = 16
NW = NC * NS
K = 200
E_PER_SUB = E // NW
CHUNKS = E_PER_SUB // K
N_CHUNK = 1000

@functools.cache
def _get_mesh():
    return plsc.VectorSubcoreMesh(core_axis_name="c", subcore_axis_name="s",
                                  num_cores=NC, num_subcores=NS)


@functools.cache
def _sc_word_gather_kernel():
    return pl.kernel(
        _sc_word_gather_body,
        mesh=_get_mesh(),
        out_type=jax.ShapeDtypeStruct((B * L, D), jnp.float32),
        scratch_types=[
            pltpu.VMEM((16,), jnp.int32),
            pltpu.VMEM((16, D), jnp.float32),
            pltpu.SemaphoreType.DMA,
        ],
        compiler_params=pltpu.CompilerParams(use_tc_tiling_on_sc=False),
    )


def _sc_word_gather_body(q_hbm, emb_hbm, out_hbm, idx_v, rows_v, sem):
    wid = lax.axis_index("s") * NC + lax.axis_index("c")
    base = wid * 16
    pltpu.sync_copy(q_hbm.at[pl.ds(base, 16)], idx_v)
    pltpu.async_copy(emb_hbm.at[idx_v], rows_v, sem).wait()
    pltpu.sync_copy(rows_v, out_hbm.at[pl.ds(base, 16)])


@functools.cache
def _sc_follow_kernel():
    return pl.kernel(
        _sc_follow_body,
        mesh=_get_mesh(),
        out_type=jax.ShapeDtypeStruct((NC, N, B), jnp.float32),
        scratch_types=[
            pltpu.VMEM((CHUNKS, K), jnp.int32),
            pltpu.VMEM((CHUNKS, K), jnp.int32),
            pltpu.VMEM((K, B), jnp.float32),
            pltpu.VMEM((K, B), jnp.float32),
            pltpu.VMEM((K, B), jnp.float32),
            pltpu.VMEM((K, B), jnp.float32),
            pltpu.VMEM((N_CHUNK, B), jnp.float32),
            pltpu.VMEM_SHARED((N, B), jnp.float32),
            pltpu.SemaphoreType.DMA,
            pltpu.SemaphoreType.DMA,
            pltpu.SemaphoreType.DMA,
            pltpu.SemaphoreType.DMA,
            pltpu.SemaphoreType.DMA,
            pltpu.SemaphoreType.DMA,
        ],
        compiler_params=pltpu.CompilerParams(use_tc_tiling_on_sc=False),
    )


def _sc_follow_body(e_hbm, sub_hbm, obj_hbm, dp_hbm, out_hbm,
                    sub_all, obj_all, rows0, rows1, dpv0, dpv1, zbuf, acc,
                    gs0, gs1, ds0, ds1, ss0, ss1):
    c = lax.axis_index("c")
    s = lax.axis_index("s")

    def _zero(i, _):
        zbuf[i, :] = jnp.zeros((B,), jnp.float32)
        return _
    lax.fori_loop(0, N_CHUNK, _zero, None)

    @pl.when(s < N // N_CHUNK)
    def _():
        pltpu.sync_copy(zbuf, acc.at[pl.ds(s * N_CHUNK, N_CHUNK)])

    wid = s * NC + c
    base = wid * E_PER_SUB
    pltpu.sync_copy(sub_hbm.at[wid], sub_all)
    pltpu.sync_copy(obj_hbm.at[wid], obj_all)
    plsc.subcore_barrier()

    def _issue(i, rows, dpv, gs, ds):
        pltpu.async_copy(e_hbm.at[sub_all.at[i]], rows, gs)
        pltpu.async_copy(dp_hbm.at[pl.ds(base + i * K, K)], dpv, ds)

    def _proc(i, rows, dpv, gs, ds, ss):
        pltpu.make_async_copy(e_hbm.at[sub_all.at[i]], rows, gs).wait()
        pltpu.make_async_copy(dp_hbm.at[pl.ds(base + i * K, K)], dpv, ds).wait()
        for j in range(K):
            rows[j, :] = rows[j, :] * dpv[j, :]
        pltpu.async_copy(rows, acc.at[obj_all.at[i]], ss, add=True)

    _issue(0, rows0, dpv0, gs0, ds0)

    def _body(i, _):
        def _phase(crows, cdpv, cgs, cds, css, nrows, ndpv, ngs, nds, nss):
            @pl.when(i > 0)
            def _():
                pltpu.make_async_copy(nrows, acc.at[obj_all.at[i - 1]],
                                      nss).wait()

            @pl.when(i + 1 < CHUNKS)
            def _():
                _issue(i + 1, nrows, ndpv, ngs, nds)
            _proc(i, crows, cdpv, cgs, cds, css)

        @pl.when(i % 2 == 0)
        def _():
            _phase(rows0, dpv0, gs0, ds0, ss0, rows1, dpv1, gs1, ds1, ss1)

        @pl.when(i % 2 == 1)
        def _():
            _phase(rows1, dpv1, gs1, ds1, ss1, rows0, dpv0, gs0, ds0, ss0)
        return _
    lax.fori_loop(0, CHUNKS, _body, None)

    pltpu.make_async_copy(rows0, acc.at[obj_all.at[CHUNKS - 1]], ss0).wait()
    plsc.subcore_barrier()

    @pl.when(s < N // N_CHUNK)
    def _():
        pltpu.sync_copy(acc.at[pl.ds(s * N_CHUNK, N_CHUNK)], zbuf)
        pltpu.sync_copy(zbuf, out_hbm.at[c, pl.ds(s * N_CHUNK, N_CHUNK)])


@functools.cache
def _sc_follow2_kernel():
    return pl.kernel(
        _sc_follow2_body,
        mesh=_get_mesh(),
        out_type=jax.ShapeDtypeStruct((NC, N, B), jnp.float32),
        scratch_types=[
            pltpu.VMEM((CHUNKS, K), jnp.int32),
            pltpu.VMEM((CHUNKS, K), jnp.int32),
            pltpu.VMEM((K, B), jnp.float32),
            pltpu.VMEM((K, B), jnp.float32),
            pltpu.VMEM((K, B), jnp.float32),
            pltpu.VMEM((K, B), jnp.float32),
            pltpu.VMEM((N_CHUNK, B), jnp.float32),
            pltpu.VMEM((N_CHUNK, B), jnp.float32),
            pltpu.VMEM_SHARED((N, B), jnp.float32),
            pltpu.VMEM_SHARED((N, B), jnp.float32),
            pltpu.SemaphoreType.DMA,
            pltpu.SemaphoreType.DMA,
            pltpu.SemaphoreType.DMA,
            pltpu.SemaphoreType.DMA,
            pltpu.SemaphoreType.DMA,
            pltpu.SemaphoreType.DMA,
        ],
        compiler_params=pltpu.CompilerParams(use_tc_tiling_on_sc=False),
    )


def _sc_follow2_body(part_hbm, sub_hbm, obj_hbm, dp_hbm, out_hbm,
                     sub_all, obj_all, rows0, rows1, dpv0, dpv1, zbuf, pbuf,
                     acc, e1_sp, gs0, gs1, ds0, ds1, ss0, ss1):
    c = lax.axis_index("c")
    s = lax.axis_index("s")

    @pl.when(s < N // N_CHUNK)
    def _():
        pltpu.sync_copy(part_hbm.at[0, pl.ds(s * N_CHUNK, N_CHUNK)], zbuf)
        pltpu.sync_copy(part_hbm.at[1, pl.ds(s * N_CHUNK, N_CHUNK)], pbuf)

        def _comb(i, _):
            v = zbuf[i, :] + pbuf[i, :]
            zbuf[i, :] = v / jnp.maximum(v, 1.0)
            return _
        lax.fori_loop(0, N_CHUNK, _comb, None)
        pltpu.sync_copy(zbuf, e1_sp.at[pl.ds(s * N_CHUNK, N_CHUNK)])

    def _zero(i, _):
        zbuf[i, :] = jnp.zeros((B,), jnp.float32)
        return _
    lax.fori_loop(0, N_CHUNK, _zero, None)

    @pl.when(s < N // N_CHUNK)
    def _():
        pltpu.sync_copy(zbuf, acc.at[pl.ds(s * N_CHUNK, N_CHUNK)])

    wid = s * NC + c
    base = wid * E_PER_SUB
    pltpu.sync_copy(sub_hbm.at[wid], sub_all)
    pltpu.sync_copy(obj_hbm.at[wid], obj_all)
    plsc.subcore_barrier()

    def _issue(i, rows, dpv, gs, ds):
        pltpu.async_copy(e1_sp.at[sub_all.at[i]], rows, gs)
        pltpu.async_copy(dp_hbm.at[pl.ds(base + i * K, K)], dpv, ds)

    def _proc(i, rows, dpv, gs, ds, ss):
        pltpu.make_async_copy(e1_sp.at[sub_all.at[i]], rows, gs).wait()
        pltpu.make_async_copy(dp_hbm.at[pl.ds(base + i * K, K)], dpv,
                              ds).wait()
        for j in range(K):
            rows[j, :] = rows[j, :] * dpv[j, :]
        pltpu.async_copy(rows, acc.at[obj_all.at[i]], ss, add=True)

    _issue(0, rows0, dpv0, gs0, ds0)

    def _body(i, _):
        def _phase(crows, cdpv, cgs, cds, css, nrows, ndpv, ngs, nds, nss):
            @pl.when(i > 0)
            def _():
                pltpu.make_async_copy(nrows, acc.at[obj_all.at[i - 1]],
                                      nss).wait()

            @pl.when(i + 1 < CHUNKS)
            def _():
                _issue(i + 1, nrows, ndpv, ngs, nds)
            _proc(i, crows, cdpv, cgs, cds, css)

        @pl.when(i % 2 == 0)
        def _():
            _phase(rows0, dpv0, gs0, ds0, ss0, rows1, dpv1, gs1, ds1, ss1)

        @pl.when(i % 2 == 1)
        def _():
            _phase(rows1, dpv1, gs1, ds1, ss1, rows0, dpv0, gs0, ds0, ss0)
        return _
    lax.fori_loop(0, CHUNKS, _body, None)
    pltpu.make_async_copy(rows0, acc.at[obj_all.at[CHUNKS - 1]], ss0).wait()
    plsc.subcore_barrier()

    @pl.when(s < N // N_CHUNK)
    def _():
        pltpu.sync_copy(acc.at[pl.ds(s * N_CHUNK, N_CHUNK)], zbuf)
        pltpu.sync_copy(zbuf, out_hbm.at[c, pl.ds(s * N_CHUNK, N_CHUNK)])


def _tc_qenc_body(qwe_ref, questions_ref, qpW_ref, qpb_ref, W0_ref, b0_ref,
                  W1_ref, b1_ref, relw_ref, qclsW_ref, qclsb_ref, es_ref,
                  cr_ref, qmask_ref, e0t_ref):
    qwe = qwe_ref[...]
    questions = questions_ref[...]
    mask = (questions != 0).astype(jnp.float32)
    lens = jnp.maximum(mask.sum(axis=1, keepdims=True), 1.0)
    qh = jnp.tanh(
        jax.lax.dot_general(qwe, qpW_ref[...],
                            (((2,), (0,)), ((), ())),
                            preferred_element_type=jnp.float32)
        + qpb_ref[...][None, :, :])
    q_emb = (qh * mask[:, :, None]).sum(axis=1) / lens

    crs = []
    for W_ref, b_ref in ((W0_ref, b0_ref), (W1_ref, b1_ref)):
        cq = jnp.tanh(
            jax.lax.dot_general(q_emb, W_ref[...],
                                (((1,), (0,)), ((), ())),
                                preferred_element_type=jnp.float32)
            + b_ref[...])
        lg = (qh * cq[:, None, :]).sum(axis=2)
        m = lg.max(axis=1, keepdims=True)
        ex = jnp.exp(lg - m)
        dist = ex / ex.sum(axis=1, keepdims=True)
        ctx = (qh * dist[:, :, None]).sum(axis=1) + cq
        crs.append(ctx * relw_ref[...])
    cr_ref[...] = jnp.concatenate(crs, axis=0)

    qm = jax.lax.dot_general(qclsW_ref[...], q_emb,
                             (((0,), (1,)), ((), ())),
                             preferred_element_type=jnp.float32)
    qmask_ref[...] = jax.nn.sigmoid(qm + qclsb_ref[...])

    e0t_ref[...] = jnp.transpose(es_ref[...], (1, 0))


def _tc_qenc(qwe3, questions, qpW, qpb, W0, b0, W1, b1, relw, qclsW, qclsb2,
             e_s):
    return pl.pallas_call(
        _tc_qenc_body,
        out_shape=(
            jax.ShapeDtypeStruct((2 * B, D), jnp.float32),
            jax.ShapeDtypeStruct((N, B), jnp.float32),
            jax.ShapeDtypeStruct((N, B), jnp.float32),
        ),
    )(qwe3, questions, qpW, qpb, W0, b0, W1, b1, relw, qclsW, qclsb2, e_s)


_DESC_TILE = 8000


def _tc_dprob_body(desc_ref, cr_ref, relb_ref, d0_ref, d1_ref):
    t = jax.lax.dot_general(desc_ref[...], cr_ref[...],
                            (((1,), (1,)), ((), ())),
                            preferred_element_type=jnp.float32)
    p = jax.nn.sigmoid(t + relb_ref[0, 0])
    d0_ref[...] = p[:, :B]
    d1_ref[...] = p[:, B:]


def _tc_dprob(desc_emb, cr, relb):
    grid = (E // _DESC_TILE,)
    return pl.pallas_call(
        _tc_dprob_body,
        grid=grid,
        in_specs=[
            pl.BlockSpec((_DESC_TILE, D), lambda i: (i, 0)),
            pl.BlockSpec((2 * B, D), lambda i: (0, 0)),
            pl.BlockSpec((1, 1), lambda i: (0, 0)),
        ],
        out_specs=(
            pl.BlockSpec((_DESC_TILE, B), lambda i: (i, 0)),
            pl.BlockSpec((_DESC_TILE, B), lambda i: (i, 0)),
        ),
        out_shape=(
            jax.ShapeDtypeStruct((E, B), jnp.float32),
            jax.ShapeDtypeStruct((E, B), jnp.float32),
        ),
    )(desc_emb, cr, relb)


def _tc_combine_body(p_ref, o_ref):
    snew = p_ref[0] + p_ref[1]
    o_ref[...] = snew / jnp.maximum(snew, 1.0)


def _tc_combine(partials):
    p = partials.reshape(NC, (N * B) // D, D)
    return pl.pallas_call(
        _tc_combine_body,
        out_shape=jax.ShapeDtypeStruct(((N * B) // D, D), jnp.float32),
    )(p).reshape(N, B)


def _tc_combine_final_body(p_ref, qm_ref, o_ref):
    snew = p_ref[0] + p_ref[1]
    res = (snew / jnp.maximum(snew, 1.0)) * qm_ref[...]
    o_ref[...] = jnp.transpose(res, (1, 0))


def _tc_combine_final(partials, qmask_t):
    return pl.pallas_call(
        _tc_combine_final_body,
        out_shape=jax.ShapeDtypeStruct((B, N), jnp.float32),
    )(partials, qmask_t)


def kernel(questions, e_s, kb_pair, word_emb, q_proj_W, q_proj_b,
           step_W0, step_b0, step_W1, step_b1,
           rel_w, rel_b, q_cls_W, q_cls_b, desc_emb):
    q_flat = questions.reshape(B * L).astype(jnp.int32)
    qwe = _sc_word_gather_kernel()(q_flat, word_emb)
    qwe3 = qwe.reshape(B, L, D)

    cr, qmask_t, e0_t = _tc_qenc(
        qwe3, questions.astype(jnp.int32),
        q_proj_W, q_proj_b.reshape(1, D),
        step_W0, step_b0.reshape(1, D),
        step_W1, step_b1.reshape(1, D),
        rel_w.reshape(1, D), q_cls_W, q_cls_b.reshape(N, 1), e_s)

    d0, d1 = _tc_dprob(desc_emb, cr, rel_b.reshape(1, 1))

    sub1 = kb_pair[:, 0].astype(jnp.int32).reshape(NW, CHUNKS, K)
    obj1 = kb_pair[:, 1].astype(jnp.int32).reshape(NW, CHUNKS, K)

    follow = _sc_follow_kernel()
    part0 = follow(e0_t, sub1, obj1, d0)
    e1_t = _tc_combine(part0)
    part1 = follow(e1_t, sub1, obj1, d1)
    return _tc_combine_final(part1, qmask_t)


# speedup vs baseline: 1.3238x; 1.3238x over previous
import functools

import jax
import jax.numpy as jnp
from jax import lax
from jax.experimental import pallas as pl
from jax.experimental.pallas import tpu as pltpu
from jax.experimental.pallas import tpu_sc as plsc

B = 16
L = 32
E = 160000
N = 10000
D = 128

NC = 2
NS = 16
NW = NC * NS
K = 200
E_PER_SUB = E // NW
CHUNKS = E_PER_SUB // K
N_CHUNK = 1000

@functools.cache
def _get_mesh():
    return plsc.VectorSubcoreMesh(core_axis_name="c", subcore_axis_name="s",
                                  num_cores=NC, num_subcores=NS)


@functools.cache
def _sc_word_gather_kernel():
    return pl.kernel(
        _sc_word_gather_body,
        mesh=_get_mesh(),
        out_type=jax.ShapeDtypeStruct((B * L, D), jnp.float32),
        scratch_types=[
            pltpu.VMEM((16,), jnp.int32),
            pltpu.VMEM((16, D), jnp.float32),
            pltpu.SemaphoreType.DMA,
        ],
        compiler_params=pltpu.CompilerParams(use_tc_tiling_on_sc=False),
    )


def _sc_word_gather_body(q_hbm, emb_hbm, out_hbm, idx_v, rows_v, sem):
    wid = lax.axis_index("s") * NC + lax.axis_index("c")
    base = wid * 16
    pltpu.sync_copy(q_hbm.at[pl.ds(base, 16)], idx_v)
    pltpu.async_copy(emb_hbm.at[idx_v], rows_v, sem).wait()
    pltpu.sync_copy(rows_v, out_hbm.at[pl.ds(base, 16)])


@functools.cache
def _sc_follow_kernel():
    return pl.kernel(
        _sc_follow_body,
        mesh=_get_mesh(),
        out_type=jax.ShapeDtypeStruct((NC, N, B), jnp.float32),
        scratch_types=[
            pltpu.VMEM((CHUNKS, K), jnp.int32),
            pltpu.VMEM((CHUNKS, K), jnp.int32),
            pltpu.VMEM((K, B), jnp.float32),
            pltpu.VMEM((K, B), jnp.float32),
            pltpu.VMEM((K // 8, 8 * B), jnp.float32),
            pltpu.VMEM((K // 8, 8 * B), jnp.float32),
            pltpu.VMEM((N_CHUNK, B), jnp.float32),
            pltpu.VMEM_SHARED((N, B), jnp.float32),
            pltpu.SemaphoreType.DMA,
            pltpu.SemaphoreType.DMA,
            pltpu.SemaphoreType.DMA,
            pltpu.SemaphoreType.DMA,
            pltpu.SemaphoreType.DMA,
            pltpu.SemaphoreType.DMA,
        ],
        compiler_params=pltpu.CompilerParams(use_tc_tiling_on_sc=False),
    )


def _sc_follow_body(e_hbm, sub_hbm, obj_hbm, dp_hbm, out_hbm,
                    sub_all, obj_all, rows0, rows1, dpv0, dpv1, zbuf, acc,
                    gs0, gs1, ds0, ds1, ss0, ss1):
    c = lax.axis_index("c")
    s = lax.axis_index("s")

    def _zero(i, _):
        zbuf[i, :] = jnp.zeros((B,), jnp.float32)
        return _
    lax.fori_loop(0, N_CHUNK, _zero, None)

    @pl.when(s < N // N_CHUNK)
    def _():
        pltpu.sync_copy(zbuf, acc.at[pl.ds(s * N_CHUNK, N_CHUNK)])

    wid = s * NC + c
    base = wid * E_PER_SUB
    pltpu.sync_copy(sub_hbm.at[wid], sub_all)
    pltpu.sync_copy(obj_hbm.at[wid], obj_all)
    plsc.subcore_barrier()

    base8 = base // 8
    K8 = K // 8

    def _issue(i, rows, dpv, gs, ds):
        pltpu.async_copy(e_hbm.at[sub_all.at[i]], rows, gs)
        pltpu.async_copy(dp_hbm.at[pl.ds(base8 + i * K8, K8)], dpv, ds)

    def _proc(i, rows, dpv, gs, ds, ss):
        pltpu.make_async_copy(e_hbm.at[sub_all.at[i]], rows, gs).wait()
        pltpu.make_async_copy(dp_hbm.at[pl.ds(base8 + i * K8, K8)], dpv,
                              ds).wait()
        for j in range(K):
            rows[j, :] = rows[j, :] * dpv[j // 8, (j % 8) * B:(j % 8 + 1) * B]
        pltpu.async_copy(rows, acc.at[obj_all.at[i]], ss, add=True)

    _issue(0, rows0, dpv0, gs0, ds0)

    def _body(i, _):
        def _phase(crows, cdpv, cgs, cds, css, nrows, ndpv, ngs, nds, nss):
            @pl.when(i > 0)
            def _():
                pltpu.make_async_copy(nrows, acc.at[obj_all.at[i - 1]],
                                      nss).wait()

            @pl.when(i + 1 < CHUNKS)
            def _():
                _issue(i + 1, nrows, ndpv, ngs, nds)
            _proc(i, crows, cdpv, cgs, cds, css)

        @pl.when(i % 2 == 0)
        def _():
            _phase(rows0, dpv0, gs0, ds0, ss0, rows1, dpv1, gs1, ds1, ss1)

        @pl.when(i % 2 == 1)
        def _():
            _phase(rows1, dpv1, gs1, ds1, ss1, rows0, dpv0, gs0, ds0, ss0)
        return _
    lax.fori_loop(0, CHUNKS, _body, None)

    pltpu.make_async_copy(rows0, acc.at[obj_all.at[CHUNKS - 1]], ss0).wait()
    plsc.subcore_barrier()

    @pl.when(s < N // N_CHUNK)
    def _():
        pltpu.sync_copy(acc.at[pl.ds(s * N_CHUNK, N_CHUNK)], zbuf)
        pltpu.sync_copy(zbuf, out_hbm.at[c, pl.ds(s * N_CHUNK, N_CHUNK)])


def _tc_qenc_body(qwe_ref, questions_ref, qpW_ref, qpb_ref, W0_ref, b0_ref,
                  W1_ref, b1_ref, relw_ref, qclsW_ref, qclsb_ref, es_ref,
                  cr_ref, qmask_ref, e0t_ref):
    qwe = qwe_ref[...]
    questions = questions_ref[...]
    mask = (questions != 0).astype(jnp.float32)
    lens = jnp.maximum(mask.sum(axis=1, keepdims=True), 1.0)
    qh = jnp.tanh(
        jax.lax.dot_general(qwe, qpW_ref[...],
                            (((2,), (0,)), ((), ())),
                            preferred_element_type=jnp.float32)
        + qpb_ref[...][None, :, :])
    q_emb = (qh * mask[:, :, None]).sum(axis=1) / lens

    crs = []
    for W_ref, b_ref in ((W0_ref, b0_ref), (W1_ref, b1_ref)):
        cq = jnp.tanh(
            jax.lax.dot_general(q_emb, W_ref[...],
                                (((1,), (0,)), ((), ())),
                                preferred_element_type=jnp.float32)
            + b_ref[...])
        lg = (qh * cq[:, None, :]).sum(axis=2)
        m = lg.max(axis=1, keepdims=True)
        ex = jnp.exp(lg - m)
        dist = ex / ex.sum(axis=1, keepdims=True)
        ctx = (qh * dist[:, :, None]).sum(axis=1) + cq
        crs.append(ctx * relw_ref[...])
    cr_ref[...] = jnp.concatenate(crs, axis=0)

    qm = jax.lax.dot_general(qclsW_ref[...], q_emb,
                             (((0,), (1,)), ((), ())),
                             preferred_element_type=jnp.float32)
    qmask_ref[...] = jax.nn.sigmoid(qm + qclsb_ref[...])

    e0t_ref[...] = jnp.transpose(es_ref[...], (1, 0))


def _tc_qenc(qwe3, questions, qpW, qpb, W0, b0, W1, b1, relw, qclsW, qclsb2,
             e_s):
    return pl.pallas_call(
        _tc_qenc_body,
        out_shape=(
            jax.ShapeDtypeStruct((2 * B, D), jnp.float32),
            jax.ShapeDtypeStruct((N, B), jnp.float32),
            jax.ShapeDtypeStruct((N, B), jnp.float32),
        ),
    )(qwe3, questions, qpW, qpb, W0, b0, W1, b1, relw, qclsW, qclsb2, e_s)


_DESC_TILE = 8000


def _tc_dprob_body(desc_ref, cr_ref, relb_ref, d0_ref, d1_ref):
    t = jax.lax.dot_general(desc_ref[...], cr_ref[...],
                            (((1,), (1,)), ((), ())),
                            preferred_element_type=jnp.float32)
    p = jax.nn.sigmoid(t + relb_ref[0, 0])
    p3 = p.reshape(_DESC_TILE // 8, 8, 2 * B)
    d0_ref[...] = jnp.concatenate([p3[:, k, :B] for k in range(8)], axis=1)
    d1_ref[...] = jnp.concatenate([p3[:, k, B:] for k in range(8)], axis=1)


def _tc_dprob(desc_emb, cr, relb):
    grid = (E // _DESC_TILE,)
    return pl.pallas_call(
        _tc_dprob_body,
        grid=grid,
        in_specs=[
            pl.BlockSpec((_DESC_TILE, D), lambda i: (i, 0)),
            pl.BlockSpec((2 * B, D), lambda i: (0, 0)),
            pl.BlockSpec((1, 1), lambda i: (0, 0)),
        ],
        out_specs=(
            pl.BlockSpec((_DESC_TILE // 8, 8 * B), lambda i: (i, 0)),
            pl.BlockSpec((_DESC_TILE // 8, 8 * B), lambda i: (i, 0)),
        ),
        out_shape=(
            jax.ShapeDtypeStruct((E // 8, 8 * B), jnp.float32),
            jax.ShapeDtypeStruct((E // 8, 8 * B), jnp.float32),
        ),
    )(desc_emb, cr, relb)


def _tc_combine_body(p_ref, o_ref):
    snew = p_ref[0] + p_ref[1]
    o_ref[...] = snew / jnp.maximum(snew, 1.0)


def _tc_combine(partials):
    p = partials.reshape(NC, (N * B) // D, D)
    return pl.pallas_call(
        _tc_combine_body,
        out_shape=jax.ShapeDtypeStruct(((N * B) // D, D), jnp.float32),
    )(p).reshape(N, B)


def _tc_combine_final_body(p_ref, qm_ref, o_ref):
    snew = p_ref[0] + p_ref[1]
    res = (snew / jnp.maximum(snew, 1.0)) * qm_ref[...]
    o_ref[...] = jnp.transpose(res, (1, 0))


def _tc_combine_final(partials, qmask_t):
    return pl.pallas_call(
        _tc_combine_final_body,
        out_shape=jax.ShapeDtypeStruct((B, N), jnp.float32),
    )(partials, qmask_t)


def kernel(questions, e_s, kb_pair, word_emb, q_proj_W, q_proj_b,
           step_W0, step_b0, step_W1, step_b1,
           rel_w, rel_b, q_cls_W, q_cls_b, desc_emb):
    q_flat = questions.reshape(B * L).astype(jnp.int32)
    qwe = _sc_word_gather_kernel()(q_flat, word_emb)
    qwe3 = qwe.reshape(B, L, D)

    cr, qmask_t, e0_t = _tc_qenc(
        qwe3, questions.astype(jnp.int32),
        q_proj_W, q_proj_b.reshape(1, D),
        step_W0, step_b0.reshape(1, D),
        step_W1, step_b1.reshape(1, D),
        rel_w.reshape(1, D), q_cls_W, q_cls_b.reshape(N, 1), e_s)

    d0, d1 = _tc_dprob(desc_emb, cr, rel_b.reshape(1, 1))

    sub1 = kb_pair[:, 0].astype(jnp.int32).reshape(NW, CHUNKS, K)
    obj1 = kb_pair[:, 1].astype(jnp.int32).reshape(NW, CHUNKS, K)

    follow = _sc_follow_kernel()
    part0 = follow(e0_t, sub1, obj1, d0)
    e1_t = _tc_combine(part0)
    part1 = follow(e1_t, sub1, obj1, d1)
    return _tc_combine_final(part1, qmask_t)


# speedup vs baseline: 1.4365x; 1.0851x over previous
import functools

import jax
import jax.numpy as jnp
from jax import lax
from jax.experimental import pallas as pl
from jax.experimental.pallas import tpu as pltpu
from jax.experimental.pallas import tpu_sc as plsc

B = 16
L = 32
E = 160000
N = 10000
D = 128

NC = 2
NS = 16
NW = NC * NS
K = 200
E_PER_SUB = E // NW
CHUNKS = E_PER_SUB // K
N_CHUNK = 1000

@functools.cache
def _get_mesh():
    return plsc.VectorSubcoreMesh(core_axis_name="c", subcore_axis_name="s",
                                  num_cores=NC, num_subcores=NS)


@functools.cache
def _sc_word_gather_kernel():
    return pl.kernel(
        _sc_word_gather_body,
        mesh=_get_mesh(),
        out_type=jax.ShapeDtypeStruct((B * L, D), jnp.float32),
        scratch_types=[
            pltpu.VMEM((16,), jnp.int32),
            pltpu.VMEM((16, D), jnp.float32),
            pltpu.SemaphoreType.DMA,
        ],
        compiler_params=pltpu.CompilerParams(use_tc_tiling_on_sc=False),
    )


def _sc_word_gather_body(q_hbm, emb_hbm, out_hbm, idx_v, rows_v, sem):
    wid = lax.axis_index("s") * NC + lax.axis_index("c")
    base = wid * 16
    pltpu.sync_copy(q_hbm.at[pl.ds(base, 16)], idx_v)
    pltpu.async_copy(emb_hbm.at[idx_v], rows_v, sem).wait()
    pltpu.sync_copy(rows_v, out_hbm.at[pl.ds(base, 16)])


@functools.cache
def _sc_follow_kernel():
    return pl.kernel(
        _sc_follow_body,
        mesh=_get_mesh(),
        out_type=jax.ShapeDtypeStruct((NC, N, B), jnp.float32),
        scratch_types=[
            pltpu.VMEM((CHUNKS, K), jnp.int32),
            pltpu.VMEM((CHUNKS, K), jnp.int32),
            pltpu.VMEM((K, B), jnp.float32),
            pltpu.VMEM((K, B), jnp.float32),
            pltpu.VMEM((K // 8, 8 * B), jnp.float32),
            pltpu.VMEM((K // 8, 8 * B), jnp.float32),
            pltpu.VMEM((N_CHUNK, B), jnp.float32),
            pltpu.VMEM_SHARED((N, B), jnp.float32),
            pltpu.SemaphoreType.DMA,
            pltpu.SemaphoreType.DMA,
            pltpu.SemaphoreType.DMA,
            pltpu.SemaphoreType.DMA,
            pltpu.SemaphoreType.DMA,
            pltpu.SemaphoreType.DMA,
        ],
        compiler_params=pltpu.CompilerParams(use_tc_tiling_on_sc=False),
    )


def _sc_follow_body(e_hbm, sub_hbm, obj_hbm, dp_hbm, out_hbm,
                    sub_all, obj_all, rows0, rows1, dpv0, dpv1, zbuf, acc,
                    gs0, gs1, ds0, ds1, ss0, ss1):
    c = lax.axis_index("c")
    s = lax.axis_index("s")

    def _zero(i, _):
        zbuf[i, :] = jnp.zeros((B,), jnp.float32)
        return _
    lax.fori_loop(0, N_CHUNK, _zero, None)

    @pl.when(s < N // N_CHUNK)
    def _():
        pltpu.sync_copy(zbuf, acc.at[pl.ds(s * N_CHUNK, N_CHUNK)])

    wid = s * NC + c
    base = wid * E_PER_SUB
    pltpu.sync_copy(sub_hbm.at[wid], sub_all)
    pltpu.sync_copy(obj_hbm.at[wid], obj_all)
    plsc.subcore_barrier()

    base8 = base // 8
    K8 = K // 8

    def _issue(i, rows, dpv, gs, ds):
        pltpu.async_copy(e_hbm.at[sub_all.at[i]], rows, gs)
        pltpu.async_copy(dp_hbm.at[pl.ds(base8 + i * K8, K8)], dpv, ds)

    def _proc(i, rows, dpv, gs, ds, ss):
        pltpu.make_async_copy(e_hbm.at[sub_all.at[i]], rows, gs).wait()
        pltpu.make_async_copy(dp_hbm.at[pl.ds(base8 + i * K8, K8)], dpv,
                              ds).wait()
        for j in range(K):
            rows[j, :] = rows[j, :] * dpv[j // 8, (j % 8) * B:(j % 8 + 1) * B]
        pltpu.async_copy(rows, acc.at[obj_all.at[i]], ss, add=True)

    _issue(0, rows0, dpv0, gs0, ds0)

    def _body(i, _):
        def _phase(crows, cdpv, cgs, cds, css, nrows, ndpv, ngs, nds, nss):
            @pl.when(i > 0)
            def _():
                pltpu.make_async_copy(nrows, acc.at[obj_all.at[i - 1]],
                                      nss).wait()

            @pl.when(i + 1 < CHUNKS)
            def _():
                _issue(i + 1, nrows, ndpv, ngs, nds)
            _proc(i, crows, cdpv, cgs, cds, css)

        @pl.when(i % 2 == 0)
        def _():
            _phase(rows0, dpv0, gs0, ds0, ss0, rows1, dpv1, gs1, ds1, ss1)

        @pl.when(i % 2 == 1)
        def _():
            _phase(rows1, dpv1, gs1, ds1, ss1, rows0, dpv0, gs0, ds0, ss0)
        return _
    lax.fori_loop(0, CHUNKS, _body, None)

    pltpu.make_async_copy(rows0, acc.at[obj_all.at[CHUNKS - 1]], ss0).wait()
    plsc.subcore_barrier()

    @pl.when(s < N // N_CHUNK)
    def _():
        pltpu.sync_copy(acc.at[pl.ds(s * N_CHUNK, N_CHUNK)], zbuf)
        pltpu.sync_copy(zbuf, out_hbm.at[c, pl.ds(s * N_CHUNK, N_CHUNK)])


def _tc_qenc_body(qwe_ref, questions_ref, qpW_ref, qpb_ref, W0_ref, b0_ref,
                  W1_ref, b1_ref, relw_ref, qclsW_ref, qclsb_ref, es_ref,
                  cr_ref, qmask_ref, e0t_ref):
    qwe = qwe_ref[...]
    questions = questions_ref[...]
    mask = (questions != 0).astype(jnp.float32)
    lens = jnp.maximum(mask.sum(axis=1, keepdims=True), 1.0)
    qh = jnp.tanh(
        jax.lax.dot_general(qwe, qpW_ref[...],
                            (((2,), (0,)), ((), ())),
                            preferred_element_type=jnp.float32)
        + qpb_ref[...][None, :, :])
    q_emb = (qh * mask[:, :, None]).sum(axis=1) / lens

    crs = []
    for W_ref, b_ref in ((W0_ref, b0_ref), (W1_ref, b1_ref)):
        cq = jnp.tanh(
            jax.lax.dot_general(q_emb, W_ref[...],
                                (((1,), (0,)), ((), ())),
                                preferred_element_type=jnp.float32)
            + b_ref[...])
        lg = (qh * cq[:, None, :]).sum(axis=2)
        m = lg.max(axis=1, keepdims=True)
        ex = jnp.exp(lg - m)
        dist = ex / ex.sum(axis=1, keepdims=True)
        ctx = (qh * dist[:, :, None]).sum(axis=1) + cq
        crs.append(ctx * relw_ref[...])
    cr_ref[...] = jnp.concatenate(crs, axis=0)

    qm = jax.lax.dot_general(qclsW_ref[...], q_emb,
                             (((0,), (1,)), ((), ())),
                             preferred_element_type=jnp.float32)
    qmask_ref[...] = jax.nn.sigmoid(qm + qclsb_ref[...])

    e0t_ref[...] = jnp.transpose(es_ref[...], (1, 0))


def _tc_qenc(qwe3, questions, qpW, qpb, W0, b0, W1, b1, relw, qclsW, qclsb2,
             e_s):
    return pl.pallas_call(
        _tc_qenc_body,
        out_shape=(
            jax.ShapeDtypeStruct((2 * B, D), jnp.float32),
            jax.ShapeDtypeStruct((N, B), jnp.float32),
            jax.ShapeDtypeStruct((N, B), jnp.float32),
        ),
    )(qwe3, questions, qpW, qpb, W0, b0, W1, b1, relw, qclsW, qclsb2, e_s)


_DESC_TILE = 8000


def _tc_dprob_body(desc_ref, cr_ref, relb_ref, d0_ref, d1_ref):
    t = jax.lax.dot_general(desc_ref[...], cr_ref[...],
                            (((1,), (1,)), ((), ())),
                            preferred_element_type=jnp.float32)
    p = jax.nn.sigmoid(t + relb_ref[0, 0])
    p3 = p.reshape(_DESC_TILE // 8, 8, 2 * B)
    for k in range(8):
        d0_ref[:, k * B:(k + 1) * B] = p3[:, k, :B]
        d1_ref[:, k * B:(k + 1) * B] = p3[:, k, B:]


def _tc_dprob(desc_emb, cr, relb):
    grid = (E // _DESC_TILE,)
    return pl.pallas_call(
        _tc_dprob_body,
        grid=grid,
        in_specs=[
            pl.BlockSpec((_DESC_TILE, D), lambda i: (i, 0)),
            pl.BlockSpec((2 * B, D), lambda i: (0, 0)),
            pl.BlockSpec((1, 1), lambda i: (0, 0)),
        ],
        out_specs=(
            pl.BlockSpec((_DESC_TILE // 8, 8 * B), lambda i: (i, 0)),
            pl.BlockSpec((_DESC_TILE // 8, 8 * B), lambda i: (i, 0)),
        ),
        out_shape=(
            jax.ShapeDtypeStruct((E // 8, 8 * B), jnp.float32),
            jax.ShapeDtypeStruct((E // 8, 8 * B), jnp.float32),
        ),
    )(desc_emb, cr, relb)


def _tc_combine_body(p_ref, o_ref):
    snew = p_ref[0] + p_ref[1]
    o_ref[...] = snew / jnp.maximum(snew, 1.0)


def _tc_combine(partials):
    p = partials.reshape(NC, (N * B) // D, D)
    return pl.pallas_call(
        _tc_combine_body,
        out_shape=jax.ShapeDtypeStruct(((N * B) // D, D), jnp.float32),
    )(p).reshape(N, B)


def _tc_combine_final_body(p_ref, qm_ref, o_ref):
    snew = p_ref[0] + p_ref[1]
    res = (snew / jnp.maximum(snew, 1.0)) * qm_ref[...]
    o_ref[...] = jnp.transpose(res, (1, 0))


def _tc_combine_final(partials, qmask_t):
    return pl.pallas_call(
        _tc_combine_final_body,
        out_shape=jax.ShapeDtypeStruct((B, N), jnp.float32),
    )(partials, qmask_t)


def kernel(questions, e_s, kb_pair, word_emb, q_proj_W, q_proj_b,
           step_W0, step_b0, step_W1, step_b1,
           rel_w, rel_b, q_cls_W, q_cls_b, desc_emb):
    q_flat = questions.reshape(B * L).astype(jnp.int32)
    qwe = _sc_word_gather_kernel()(q_flat, word_emb)
    qwe3 = qwe.reshape(B, L, D)

    cr, qmask_t, e0_t = _tc_qenc(
        qwe3, questions.astype(jnp.int32),
        q_proj_W, q_proj_b.reshape(1, D),
        step_W0, step_b0.reshape(1, D),
        step_W1, step_b1.reshape(1, D),
        rel_w.reshape(1, D), q_cls_W, q_cls_b.reshape(N, 1), e_s)

    d0, d1 = _tc_dprob(desc_emb, cr, rel_b.reshape(1, 1))

    sub1 = kb_pair[:, 0].astype(jnp.int32).reshape(NW, CHUNKS, K)
    obj1 = kb_pair[:, 1].astype(jnp.int32).reshape(NW, CHUNKS, K)

    follow = _sc_follow_kernel()
    part0 = follow(e0_t, sub1, obj1, d0)
    e1_t = _tc_combine(part0)
    part1 = follow(e1_t, sub1, obj1, d1)
    return _tc_combine_final(part1, qmask_t)
